# baseline (speedup 1.0000x reference)
import functools

import jax
import jax.numpy as jnp
from jax import lax
from jax.experimental import pallas as pl
from jax.experimental.pallas import tpu as pltpu
from jax.experimental.pallas import tpu_sc as plsc

_NB = 31
_L = 16
_NW = 32
_CHUNK = 32768
_UNROLL = 8

_mesh = plsc.VectorSubcoreMesh(core_axis_name="c", subcore_axis_name="s")


def _wid():
    return lax.axis_index("c") * 16 + lax.axis_index("s")


def _params_kernel(sc_ref, tcs_ref, st_ref, e_ref, p_ref):
    x = sc_ref[...]
    col = lax.broadcasted_iota(jnp.int32, x.shape, 1)
    comp = (col // _L) % 4
    mn = jnp.minimum(jnp.min(jnp.where(comp == 0, x, jnp.inf)), tcs_ref[0])
    mx = jnp.maximum(jnp.max(jnp.where(comp == 1, x, -jnp.inf)), tcs_ref[1])
    s = jnp.sum(jnp.where(comp == 2, x, 0.0)) + tcs_ref[2]
    ss = jnp.sum(jnp.where(comp == 3, x, 0.0)) + tcs_ref[3]
    st_ref[0] = mn
    st_ref[1] = mx
    st_ref[2] = s
    st_ref[3] = ss
    r0 = jnp.where(mx == mn, mn - 0.5, mn)
    r1 = jnp.where(mx == mn, mx + 0.5, mx)
    span = r1 - r0
    for j in range(_NB):
        e_ref[j] = r0 + span * jnp.float32(j / _NB)
    e_ref[_NB] = r1
    scale = jnp.float32(_NB) / span
    for l in range(_L):
        p_ref[l] = r0
        p_ref[_L + l] = scale


def _tc_stats_kernel(x_ref, o_ref):
    i = pl.program_id(0)
    x = x_ref[...]
    mn = jnp.min(x)
    mx = jnp.max(x)
    s = jnp.sum(x)
    ss = jnp.sum(x * x)

    @pl.when(i == 0)
    def _init():
        o_ref[0] = mn
        o_ref[1] = mx
        o_ref[2] = s
        o_ref[3] = ss

    @pl.when(i != 0)
    def _acc():
        o_ref[0] = jnp.minimum(o_ref[0], mn)
        o_ref[1] = jnp.maximum(o_ref[1], mx)
        o_ref[2] = o_ref[2] + s
        o_ref[3] = o_ref[3] + ss


def _stats_body(x_hbm, out_hbm, buf0, buf1, stats_v, sem0, sem1, *, off, npw,
                nchunks):
    base = off + _wid() * npw
    bufs = (buf0, buf1)
    sems = (sem0, sem1)

    def copy(c):
        return pltpu.make_async_copy(
            x_hbm.at[pl.ds(base + c * _CHUNK, _CHUNK)], bufs[c % 2], sems[c % 2])

    copy(0).start()
    mn = jnp.full((_L,), jnp.inf, jnp.float32)
    mx = jnp.full((_L,), -jnp.inf, jnp.float32)
    s = jnp.zeros((_L,), jnp.float32)
    ss = jnp.zeros((_L,), jnp.float32)
    carry = (mn, mx, s, ss)

    for c in range(nchunks):
        if c + 1 < nchunks:
            copy(c + 1).start()
        copy(c).wait()
        buf = bufs[c % 2]

        def body(i, carry, buf=buf):
            mn, mx, s, ss = carry
            for k in range(_UNROLL):
                v = buf[pl.ds((i * _UNROLL + k) * _L, _L)]
                mn = jnp.minimum(mn, v)
                mx = jnp.maximum(mx, v)
                s = s + v
                ss = ss + v * v
            return (mn, mx, s, ss)

        carry = lax.fori_loop(0, _CHUNK // (_UNROLL * _L), body, carry)

    mn, mx, s, ss = carry
    stats_v[pl.ds(0, _L)] = mn
    stats_v[pl.ds(_L, _L)] = mx
    stats_v[pl.ds(2 * _L, _L)] = s
    stats_v[pl.ds(3 * _L, _L)] = ss
    pltpu.sync_copy(stats_v, out_hbm.at[pl.ds(_wid() * 4 * _L, 4 * _L)])


def _tc_count_kernel(e_ref, x_ref, o_ref):
    i = pl.program_id(0)
    x = x_ref[...]

    @pl.when(i == 0)
    def _init():
        for j in range(_NB + 1):
            o_ref[j] = 0.0

    for j in range(1, _NB):
        cnt = jnp.sum((x >= e_ref[j]).astype(jnp.float32))
        o_ref[j] = o_ref[j] + cnt


def _hist_body(x_hbm, params_hbm, out_hbm, buf0, buf1, params_v, *scratch,
               off, npw, nchunks):
    hists = scratch[:_UNROLL]
    sem0, sem1 = scratch[_UNROLL:]
    base = off + _wid() * npw
    bufs = (buf0, buf1)

    def copy(c):
        return pltpu.make_async_copy(
            x_hbm.at[pl.ds(base + c * _CHUNK, _CHUNK)], bufs[c % 2], sems[c % 2])

    sems = (sem0, sem1)
    copy(0).start()
    pltpu.sync_copy(params_hbm, params_v)
    r0 = params_v[pl.ds(0, _L)]
    scl = params_v[pl.ds(_L, _L)]
    lane = lax.iota(jnp.int32, _L)
    ones = jnp.full((_L,), 1.0, jnp.float32)
    zero = jnp.zeros((_L,), jnp.float32)

    def zbody(i, _):
        for h in hists:
            h[pl.ds(i * _L, _L)] = zero
        return 0

    lax.fori_loop(0, 32, zbody, 0)

    for c in range(nchunks):
        if c + 1 < nchunks:
            copy(c + 1).start()
        copy(c).wait()
        buf = bufs[c % 2]

        def body(i, _, buf=buf):
            for k in range(_UNROLL):
                v = buf[pl.ds((i * _UNROLL + k) * _L, _L)]
                t = (v - r0) * scl
                ix = t.astype(jnp.int32)
                addr = ix * _L + lane
                plsc.addupdate_scatter(hists[k], [addr], ones)
            return 0

        lax.fori_loop(0, _CHUNK // (_UNROLL * _L), body, 0)

    def mbody(i, _):
        acc = hists[0][pl.ds(i * _L, _L)]
        for h in hists[1:]:
            acc = acc + h[pl.ds(i * _L, _L)]
        hists[0][pl.ds(i * _L, _L)] = acc
        return 0

    lax.fori_loop(0, 32, mbody, 0)
    pltpu.sync_copy(hists[0], out_hbm.at[pl.ds(_wid() * 32 * _L, 32 * _L)])


_SC_CHUNKS_PER_WORKER = 4
_SC_STATS_CHUNKS_PER_WORKER = 7
_TC_BLOCK_ROWS = 4096
_TC_LANES = 128


def kernel(array):
    n = array.size
    npw = n // _NW
    nchunks = npw // _CHUNK

    npw_st = _SC_STATS_CHUNKS_PER_WORKER * _CHUNK
    n_st_sc = _NW * npw_st
    n_st_tc = n - n_st_sc

    stats_call = pl.kernel(
        functools.partial(
            _stats_body, off=n_st_tc, npw=npw_st,
            nchunks=_SC_STATS_CHUNKS_PER_WORKER),
        out_type=jax.ShapeDtypeStruct((_NW * 4 * _L,), jnp.float32),
        mesh=_mesh,
        compiler_params=pltpu.CompilerParams(needs_layout_passes=False),
        scratch_types=[
            pltpu.VMEM((_CHUNK,), jnp.float32),
            pltpu.VMEM((_CHUNK,), jnp.float32),
            pltpu.VMEM((4 * _L,), jnp.float32),
            pltpu.SemaphoreType.DMA,
            pltpu.SemaphoreType.DMA,
        ],
    )
    stats = stats_call(array).reshape(_NW, 4, _L)

    x2 = array.reshape(-1, _TC_LANES)
    grid_st = n_st_tc // (_TC_BLOCK_ROWS * _TC_LANES)
    tc_stats = pl.pallas_call(
        _tc_stats_kernel,
        grid=(grid_st,),
        in_specs=[pl.BlockSpec((_TC_BLOCK_ROWS, _TC_LANES), lambda i: (i, 0))],
        out_specs=pl.BlockSpec(memory_space=pltpu.SMEM),
        out_shape=jax.ShapeDtypeStruct((4,), jnp.float32),
    )(x2)

    mn = jnp.minimum(stats[:, 0, :].min(), tc_stats[0])
    mx = jnp.maximum(stats[:, 1, :].max(), tc_stats[1])
    s = stats[:, 2, :].sum() + tc_stats[2]
    ss = stats[:, 3, :].sum() + tc_stats[3]
    num = jnp.asarray(n, jnp.int32)

    r0 = jnp.where(mx == mn, mn - 0.5, mn)
    r1 = jnp.where(mx == mn, mx + 0.5, mx)
    span = r1 - r0
    edges = r0 + span * (lax.iota(jnp.float32, _NB + 1) * (1.0 / _NB))
    edges = edges.at[_NB].set(r1)
    scale = jnp.float32(_NB) / span
    params = jnp.concatenate([
        jnp.full((_L,), r0, jnp.float32),
        jnp.full((_L,), scale, jnp.float32),
    ])

    npw_sc = _SC_CHUNKS_PER_WORKER * _CHUNK
    n_sc = _NW * npw_sc
    n_tc = n - n_sc

    hist_call = pl.kernel(
        functools.partial(
            _hist_body, off=n_tc, npw=npw_sc, nchunks=_SC_CHUNKS_PER_WORKER),
        out_type=jax.ShapeDtypeStruct((_NW * 32 * _L,), jnp.float32),
        mesh=_mesh,
        compiler_params=pltpu.CompilerParams(needs_layout_passes=False),
        scratch_types=[
            pltpu.VMEM((_CHUNK,), jnp.float32),
            pltpu.VMEM((_CHUNK,), jnp.float32),
            pltpu.VMEM((2 * _L,), jnp.float32),
        ] + [pltpu.VMEM((32 * _L,), jnp.float32) for _ in range(_UNROLL)] + [
            pltpu.SemaphoreType.DMA,
            pltpu.SemaphoreType.DMA,
        ],
    )
    tables = hist_call(array, params).reshape(_NW, 32, _L)
    rows = tables.sum(axis=(0, 2))
    counts_sc = rows[:_NB].at[_NB - 1].add(rows[_NB])

    x2 = array.reshape(-1, _TC_LANES)
    grid = n_tc // (_TC_BLOCK_ROWS * _TC_LANES)
    g = pl.pallas_call(
        _tc_count_kernel,
        grid=(grid,),
        in_specs=[
            pl.BlockSpec(memory_space=pltpu.SMEM),
            pl.BlockSpec((_TC_BLOCK_ROWS, _TC_LANES), lambda i: (i, 0)),
        ],
        out_specs=pl.BlockSpec(memory_space=pltpu.SMEM),
        out_shape=jax.ShapeDtypeStruct((_NB + 1,), jnp.float32),
    )(edges, x2)
    gfull = g.at[0].set(jnp.float32(n_tc)).at[_NB].set(0.0)
    counts_tc = gfull[:_NB] - gfull[1:_NB + 1]

    counts = counts_sc + counts_tc
    return (mn, mx, num, s, ss, edges, counts)


# speedup vs baseline: 1.0652x; 1.0652x over previous
import functools

import jax
import jax.numpy as jnp
from jax import lax
from jax.experimental import pallas as pl
from jax.experimental.pallas import tpu as pltpu
from jax.experimental.pallas import tpu_sc as plsc

_NB = 31
_L = 16
_NW = 32
_CHUNK = 32768
_UNROLL = 8

_mesh = plsc.VectorSubcoreMesh(core_axis_name="c", subcore_axis_name="s")


def _wid():
    return lax.axis_index("c") * 16 + lax.axis_index("s")


def _params_kernel(sc_ref, tcs_ref, st_ref, e_ref, p_ref):
    x = sc_ref[...]
    col = lax.broadcasted_iota(jnp.int32, x.shape, 1)
    comp = (col // _L) % 4
    mn = jnp.minimum(jnp.min(jnp.where(comp == 0, x, jnp.inf)), tcs_ref[0])
    mx = jnp.maximum(jnp.max(jnp.where(comp == 1, x, -jnp.inf)), tcs_ref[1])
    s = jnp.sum(jnp.where(comp == 2, x, 0.0)) + tcs_ref[2]
    ss = jnp.sum(jnp.where(comp == 3, x, 0.0)) + tcs_ref[3]
    st_ref[0] = mn
    st_ref[1] = mx
    st_ref[2] = s
    st_ref[3] = ss
    r0 = jnp.where(mx == mn, mn - 0.5, mn)
    r1 = jnp.where(mx == mn, mx + 0.5, mx)
    span = r1 - r0
    for j in range(_NB):
        e_ref[j] = r0 + span * jnp.float32(j / _NB)
    e_ref[_NB] = r1
    scale = jnp.float32(_NB) / span
    for l in range(_L):
        p_ref[l] = r0
        p_ref[_L + l] = scale


def _tc_stats_kernel(x_ref, o_ref):
    i = pl.program_id(0)
    x = x_ref[...]
    mn = jnp.min(x)
    mx = jnp.max(x)
    s = jnp.sum(x)
    ss = jnp.sum(x * x)

    @pl.when(i == 0)
    def _init():
        o_ref[0] = mn
        o_ref[1] = mx
        o_ref[2] = s
        o_ref[3] = ss

    @pl.when(i != 0)
    def _acc():
        o_ref[0] = jnp.minimum(o_ref[0], mn)
        o_ref[1] = jnp.maximum(o_ref[1], mx)
        o_ref[2] = o_ref[2] + s
        o_ref[3] = o_ref[3] + ss


def _stats_body(x_hbm, out_hbm, buf0, buf1, stats_v, sem0, sem1, *, off, npw,
                nchunks):
    base = off + _wid() * npw
    bufs = (buf0, buf1)
    sems = (sem0, sem1)

    def copy(c):
        return pltpu.make_async_copy(
            x_hbm.at[pl.ds(base + c * _CHUNK, _CHUNK)], bufs[c % 2], sems[c % 2])

    copy(0).start()
    mn = jnp.full((_L,), jnp.inf, jnp.float32)
    mx = jnp.full((_L,), -jnp.inf, jnp.float32)
    s = jnp.zeros((_L,), jnp.float32)
    ss = jnp.zeros((_L,), jnp.float32)
    carry = (mn, mx, s, ss)

    for c in range(nchunks):
        if c + 1 < nchunks:
            copy(c + 1).start()
        copy(c).wait()
        buf = bufs[c % 2]

        def body(i, carry, buf=buf):
            mn, mx, s, ss = carry
            for k in range(_UNROLL):
                v = buf[pl.ds((i * _UNROLL + k) * _L, _L)]
                mn = jnp.minimum(mn, v)
                mx = jnp.maximum(mx, v)
                s = s + v
                ss = ss + v * v
            return (mn, mx, s, ss)

        carry = lax.fori_loop(0, _CHUNK // (_UNROLL * _L), body, carry)

    mn, mx, s, ss = carry
    stats_v[pl.ds(0, _L)] = mn
    stats_v[pl.ds(_L, _L)] = mx
    stats_v[pl.ds(2 * _L, _L)] = s
    stats_v[pl.ds(3 * _L, _L)] = ss
    pltpu.sync_copy(stats_v, out_hbm.at[pl.ds(_wid() * 4 * _L, 4 * _L)])


def _tc_count_kernel(e_ref, x_ref, o_ref):
    i = pl.program_id(0)
    x = x_ref[...]

    @pl.when(i == 0)
    def _init():
        for j in range(_NB + 1):
            o_ref[j] = 0.0

    for j in range(1, _NB):
        cnt = jnp.sum((x >= e_ref[j]).astype(jnp.float32))
        o_ref[j] = o_ref[j] + cnt


def _hist_body(x_hbm, params_hbm, out_hbm, buf0, buf1, params_v, *scratch,
               off, npw, nchunks):
    hists = scratch[:_UNROLL]
    sem0, sem1 = scratch[_UNROLL:]
    base = off + _wid() * npw
    bufs = (buf0, buf1)

    def copy(c):
        return pltpu.make_async_copy(
            x_hbm.at[pl.ds(base + c * _CHUNK, _CHUNK)], bufs[c % 2], sems[c % 2])

    sems = (sem0, sem1)
    copy(0).start()
    pltpu.sync_copy(params_hbm, params_v)
    r0 = params_v[pl.ds(0, _L)]
    scl = params_v[pl.ds(_L, _L)]
    lane = lax.iota(jnp.int32, _L)
    ones = jnp.full((_L,), 1.0, jnp.float32)
    zero = jnp.zeros((_L,), jnp.float32)

    def zbody(i, _):
        for h in hists:
            h[pl.ds(i * _L, _L)] = zero
        return 0

    lax.fori_loop(0, 32, zbody, 0)

    for c in range(nchunks):
        if c + 1 < nchunks:
            copy(c + 1).start()
        copy(c).wait()
        buf = bufs[c % 2]

        def body(i, _, buf=buf):
            for k in range(_UNROLL):
                v = buf[pl.ds((i * _UNROLL + k) * _L, _L)]
                t = (v - r0) * scl
                ix = t.astype(jnp.int32)
                addr = ix * _L + lane
                plsc.addupdate_scatter(hists[k], [addr], ones)
            return 0

        lax.fori_loop(0, _CHUNK // (_UNROLL * _L), body, 0)

    def mbody(i, _):
        acc = hists[0][pl.ds(i * _L, _L)]
        for h in hists[1:]:
            acc = acc + h[pl.ds(i * _L, _L)]
        hists[0][pl.ds(i * _L, _L)] = acc
        return 0

    lax.fori_loop(0, 32, mbody, 0)
    pltpu.sync_copy(hists[0], out_hbm.at[pl.ds(_wid() * 32 * _L, 32 * _L)])


_SC_CHUNKS_PER_WORKER = 5
_SC_STATS_CHUNKS_PER_WORKER = 7
_TC_BLOCK_ROWS = 8192
_TC_LANES = 128


def kernel(array):
    n = array.size
    npw = n // _NW
    nchunks = npw // _CHUNK

    npw_st = _SC_STATS_CHUNKS_PER_WORKER * _CHUNK
    n_st_sc = _NW * npw_st
    n_st_tc = n - n_st_sc

    stats_call = pl.kernel(
        functools.partial(
            _stats_body, off=n_st_tc, npw=npw_st,
            nchunks=_SC_STATS_CHUNKS_PER_WORKER),
        out_type=jax.ShapeDtypeStruct((_NW * 4 * _L,), jnp.float32),
        mesh=_mesh,
        compiler_params=pltpu.CompilerParams(needs_layout_passes=False),
        scratch_types=[
            pltpu.VMEM((_CHUNK,), jnp.float32),
            pltpu.VMEM((_CHUNK,), jnp.float32),
            pltpu.VMEM((4 * _L,), jnp.float32),
            pltpu.SemaphoreType.DMA,
            pltpu.SemaphoreType.DMA,
        ],
    )
    stats = stats_call(array).reshape(_NW, 4, _L)

    x2 = array.reshape(-1, _TC_LANES)
    grid_st = n_st_tc // (_TC_BLOCK_ROWS * _TC_LANES)
    tc_stats = pl.pallas_call(
        _tc_stats_kernel,
        grid=(grid_st,),
        in_specs=[pl.BlockSpec((_TC_BLOCK_ROWS, _TC_LANES), lambda i: (i, 0))],
        out_specs=pl.BlockSpec(memory_space=pltpu.SMEM),
        out_shape=jax.ShapeDtypeStruct((4,), jnp.float32),
    )(x2)

    mn = jnp.minimum(stats[:, 0, :].min(), tc_stats[0])
    mx = jnp.maximum(stats[:, 1, :].max(), tc_stats[1])
    s = stats[:, 2, :].sum() + tc_stats[2]
    ss = stats[:, 3, :].sum() + tc_stats[3]
    num = jnp.asarray(n, jnp.int32)

    r0 = jnp.where(mx == mn, mn - 0.5, mn)
    r1 = jnp.where(mx == mn, mx + 0.5, mx)
    span = r1 - r0
    edges = r0 + span * (lax.iota(jnp.float32, _NB + 1) * (1.0 / _NB))
    edges = edges.at[_NB].set(r1)
    scale = jnp.float32(_NB) / span
    params = jnp.concatenate([
        jnp.full((_L,), r0, jnp.float32),
        jnp.full((_L,), scale, jnp.float32),
    ])

    npw_sc = _SC_CHUNKS_PER_WORKER * _CHUNK
    n_sc = _NW * npw_sc
    n_tc = n - n_sc

    hist_call = pl.kernel(
        functools.partial(
            _hist_body, off=n_tc, npw=npw_sc, nchunks=_SC_CHUNKS_PER_WORKER),
        out_type=jax.ShapeDtypeStruct((_NW * 32 * _L,), jnp.float32),
        mesh=_mesh,
        compiler_params=pltpu.CompilerParams(needs_layout_passes=False),
        scratch_types=[
            pltpu.VMEM((_CHUNK,), jnp.float32),
            pltpu.VMEM((_CHUNK,), jnp.float32),
            pltpu.VMEM((2 * _L,), jnp.float32),
        ] + [pltpu.VMEM((32 * _L,), jnp.float32) for _ in range(_UNROLL)] + [
            pltpu.SemaphoreType.DMA,
            pltpu.SemaphoreType.DMA,
        ],
    )
    tables = hist_call(array, params).reshape(_NW, 32, _L)
    rows = tables.sum(axis=(0, 2))
    counts_sc = rows[:_NB].at[_NB - 1].add(rows[_NB])

    x2 = array.reshape(-1, _TC_LANES)
    grid = n_tc // (_TC_BLOCK_ROWS * _TC_LANES)
    g = pl.pallas_call(
        _tc_count_kernel,
        grid=(grid,),
        in_specs=[
            pl.BlockSpec(memory_space=pltpu.SMEM),
            pl.BlockSpec((_TC_BLOCK_ROWS, _TC_LANES), lambda i: (i, 0)),
        ],
        out_specs=pl.BlockSpec(memory_space=pltpu.SMEM),
        out_shape=jax.ShapeDtypeStruct((_NB + 1,), jnp.float32),
    )(edges, x2)
    gfull = g.at[0].set(jnp.float32(n_tc)).at[_NB].set(0.0)
    counts_tc = gfull[:_NB] - gfull[1:_NB + 1]

    counts = counts_sc + counts_tc
    return (mn, mx, num, s, ss, edges, counts)
